# Initial kernel scaffold; baseline (speedup 1.0000x reference)
import jax
import jax.numpy as jnp
from jax.experimental import pallas as pl


def kernel(h, W):
    raise NotImplementedError("write your pallas kernel here")


# speedup vs baseline: 8.9647x; 8.9647x over previous
import jax
import jax.numpy as jnp
from jax.experimental import pallas as pl
from jax.experimental.pallas import tpu as pltpu

_NS = 4
_SD = 128
_K = 16
_OFFSETS = (1, 16, 256, 4096)
_BETA = 0.25


def _vq_block_kernel(x_ref, w_ref, wsq_ref, zq_ref, ids_ref, loss_ref):
    x = x_ref[...]
    bt = x.shape[0]
    ids_packed = jnp.zeros((bt,), dtype=jnp.int32)
    loss = jnp.float32(0.0)
    for i in range(_NS):
        xi = x[:, i * _SD:(i + 1) * _SD]
        wi = w_ref[i]
        dot = jax.lax.dot_general(
            xi, wi, (((1,), (1,)), ((), ())),
            preferred_element_type=jnp.float32)
        flat_sq = jnp.sum(xi * xi, axis=1, keepdims=True)
        dist = flat_sq + wsq_ref[i][None, :] - 2.0 * dot
        dmin = jnp.min(dist, axis=1, keepdims=True)
        lane = jax.lax.broadcasted_iota(jnp.int32, (bt, _K), 1)
        ids = jnp.min(jnp.where(dist == dmin, lane, _K), axis=1)
        onehot = (lane == ids[:, None]).astype(jnp.float32)
        zq = jax.lax.dot_general(
            onehot, wi, (((1,), (0,)), ((), ())),
            preferred_element_type=jnp.float32)
        zq_ref[:, i * _SD:(i + 1) * _SD] = zq
        diff = zq - xi
        loss = loss + jnp.sum(diff * diff)
        ids_packed = ids_packed + ids * jnp.int32(_OFFSETS[i])
    ids_ref[0, 0, :] = ids_packed

    @pl.when(pl.program_id(0) == 0)
    def _init():
        loss_ref[...] = jnp.zeros((1, 1), jnp.float32)

    loss_ref[...] += loss.reshape(1, 1)


def kernel(h, W):
    Bq, Nq, Dq = h.shape
    tokens = Bq * Nq
    bt = 2048
    grid = tokens // bt
    hf = h.reshape(tokens, Dq)
    W_sq = jnp.sum(W * W, axis=2)

    zq, ids3, loss = pl.pallas_call(
        _vq_block_kernel,
        grid=(grid,),
        in_specs=[
            pl.BlockSpec((bt, Dq), lambda i: (i, 0)),
            pl.BlockSpec((_NS, _K, _SD), lambda i: (0, 0, 0)),
            pl.BlockSpec((_NS, _K), lambda i: (0, 0)),
        ],
        out_specs=[
            pl.BlockSpec((bt, Dq), lambda i: (i, 0)),
            pl.BlockSpec((1, 1, bt), lambda i: (i, 0, 0)),
            pl.BlockSpec((1, 1), lambda i: (0, 0)),
        ],
        out_shape=[
            jax.ShapeDtypeStruct((tokens, Dq), jnp.float32),
            jax.ShapeDtypeStruct((grid, 1, bt), jnp.int32),
            jax.ShapeDtypeStruct((1, 1), jnp.float32),
        ],
        compiler_params=pltpu.CompilerParams(
            dimension_semantics=("arbitrary",)),
    )(hf, W, W_sq)

    z_q = zq.reshape(Bq, Nq, Dq)
    ids_packed = ids3.reshape(Bq, Nq)
    n_elems = jnp.float32(tokens * _SD)
    vq_total = (loss[0, 0] * jnp.float32(1.0 + _BETA)) / n_elems
    return (z_q, ids_packed, vq_total)


# speedup vs baseline: 12.8652x; 1.4351x over previous
import jax
import jax.numpy as jnp
from jax.experimental import pallas as pl
from jax.experimental.pallas import tpu as pltpu

_NS = 4
_SD = 128
_K = 16
_OFFSETS = (1, 16, 256, 4096)
_BETA = 0.25


def _vq_block_kernel(x_ref, w_ref, wsq_ref, zq_ref, ids_ref, loss_ref):
    x = x_ref[...]
    bt = x.shape[0]
    slane = jax.lax.broadcasted_iota(jnp.int32, (_K, bt), 0)
    packed_row = jnp.zeros((1, bt), dtype=jnp.int32)
    loss = jnp.float32(0.0)
    for i in range(_NS):
        xi = x[:, i * _SD:(i + 1) * _SD]
        wi = w_ref[i]
        dot = jax.lax.dot_general(
            xi, wi, (((1,), (1,)), ((), ())),
            preferred_element_type=jnp.float32)
        flat_sq = jnp.sum(xi * xi, axis=1, keepdims=True)
        score = (flat_sq + wsq_ref[i][None, :]) - 2.0 * dot
        score_t = jnp.transpose(score)
        smin = jnp.min(score_t, axis=0, keepdims=True)
        ids_row = jnp.min(jnp.where(score_t == smin, slane, _K),
                          axis=0, keepdims=True)
        onehot_t = (slane == ids_row).astype(jnp.float32)
        zq = jax.lax.dot_general(
            onehot_t, wi, (((0,), (0,)), ((), ())),
            preferred_element_type=jnp.float32)
        zq_ref[:, i * _SD:(i + 1) * _SD] = zq
        loss = loss + jnp.sum(smin)
        packed_row = packed_row + ids_row * jnp.int32(_OFFSETS[i])
    ids_ref[0, :, :] = packed_row

    @pl.when(pl.program_id(0) == 0)
    def _init():
        loss_ref[...] = jnp.zeros((1, 1), jnp.float32)

    loss_ref[...] += loss.reshape(1, 1)


def kernel(h, W):
    Bq, Nq, Dq = h.shape
    tokens = Bq * Nq
    bt = 2048
    grid = tokens // bt
    hf = h.reshape(tokens, Dq)
    W_sq = jnp.sum(W * W, axis=2)

    zq, ids3, loss = pl.pallas_call(
        _vq_block_kernel,
        grid=(grid,),
        in_specs=[
            pl.BlockSpec((bt, Dq), lambda i: (i, 0)),
            pl.BlockSpec((_NS, _K, _SD), lambda i: (0, 0, 0)),
            pl.BlockSpec((_NS, _K), lambda i: (0, 0)),
        ],
        out_specs=[
            pl.BlockSpec((bt, Dq), lambda i: (i, 0)),
            pl.BlockSpec((1, 1, bt), lambda i: (i, 0, 0)),
            pl.BlockSpec((1, 1), lambda i: (0, 0)),
        ],
        out_shape=[
            jax.ShapeDtypeStruct((tokens, Dq), jnp.float32),
            jax.ShapeDtypeStruct((grid, 1, bt), jnp.int32),
            jax.ShapeDtypeStruct((1, 1), jnp.float32),
        ],
        compiler_params=pltpu.CompilerParams(
            dimension_semantics=("arbitrary",)),
    )(hf, W, W_sq)

    z_q = zq.reshape(Bq, Nq, Dq)
    ids_packed = ids3.reshape(Bq, Nq)
    n_elems = jnp.float32(tokens * _SD)
    vq_total = (loss[0, 0] * jnp.float32(1.0 + _BETA)) / n_elems
    return (z_q, ids_packed, vq_total)


# speedup vs baseline: 14.5098x; 1.1278x over previous
import jax
import jax.numpy as jnp
from jax.experimental import pallas as pl
from jax.experimental.pallas import tpu as pltpu

_NS = 4
_SD = 128
_K = 16
_OFFSETS = (1, 16, 256, 4096)
_BETA = 0.25


def _vq_block_kernel(x_ref, w_ref, wsq_ref, zq_ref, ids_ref, loss_ref):
    x = x_ref[...]
    bt = x.shape[0]
    slane = jax.lax.broadcasted_iota(jnp.int32, (_K, bt), 0)
    packed_row = jnp.zeros((1, bt), dtype=jnp.int32)
    loss = jnp.float32(0.0)
    for i in range(_NS):
        xi = x[:, i * _SD:(i + 1) * _SD]
        wi = w_ref[i]
        dot = jax.lax.dot_general(
            xi, wi, (((1,), (1,)), ((), ())),
            preferred_element_type=jnp.float32)
        flat_sq = jnp.sum(xi * xi, axis=1, keepdims=True)
        score = (flat_sq + wsq_ref[i][None, :]) - 2.0 * dot
        score_t = jnp.transpose(score)
        smin = jnp.min(score_t, axis=0, keepdims=True)
        ids_row = jnp.min(jnp.where(score_t == smin, slane, _K),
                          axis=0, keepdims=True)
        onehot_t = (slane == ids_row).astype(jnp.float32)
        zq = jax.lax.dot_general(
            onehot_t, wi, (((0,), (0,)), ((), ())),
            preferred_element_type=jnp.float32)
        zq_ref[:, i * _SD:(i + 1) * _SD] = zq
        loss = loss + jnp.sum(smin)
        packed_row = packed_row + ids_row * jnp.int32(_OFFSETS[i])
    ids_ref[0, :, :] = packed_row

    @pl.when(pl.program_id(0) == 0)
    def _init():
        loss_ref[...] = jnp.zeros((1, 1), jnp.float32)

    loss_ref[...] += loss.reshape(1, 1)


def kernel(h, W):
    Bq, Nq, Dq = h.shape
    tokens = Bq * Nq
    bt = 4096
    grid = tokens // bt
    hf = h.reshape(tokens, Dq)
    W_sq = jnp.sum(W * W, axis=2)

    zq, ids3, loss = pl.pallas_call(
        _vq_block_kernel,
        grid=(grid,),
        in_specs=[
            pl.BlockSpec((bt, Dq), lambda i: (i, 0)),
            pl.BlockSpec((_NS, _K, _SD), lambda i: (0, 0, 0)),
            pl.BlockSpec((_NS, _K), lambda i: (0, 0)),
        ],
        out_specs=[
            pl.BlockSpec((bt, Dq), lambda i: (i, 0)),
            pl.BlockSpec((1, 1, bt), lambda i: (i, 0, 0)),
            pl.BlockSpec((1, 1), lambda i: (0, 0)),
        ],
        out_shape=[
            jax.ShapeDtypeStruct((tokens, Dq), jnp.float32),
            jax.ShapeDtypeStruct((grid, 1, bt), jnp.int32),
            jax.ShapeDtypeStruct((1, 1), jnp.float32),
        ],
        compiler_params=pltpu.CompilerParams(
            dimension_semantics=("arbitrary",)),
    )(hf, W, W_sq)

    z_q = zq.reshape(Bq, Nq, Dq)
    ids_packed = ids3.reshape(Bq, Nq)
    n_elems = jnp.float32(tokens * _SD)
    vq_total = (loss[0, 0] * jnp.float32(1.0 + _BETA)) / n_elems
    return (z_q, ids_packed, vq_total)


# speedup vs baseline: 15.0608x; 1.0380x over previous
import jax
import jax.numpy as jnp
from jax.experimental import pallas as pl
from jax.experimental.pallas import tpu as pltpu

_NS = 4
_SD = 128
_K = 16
_OFFSETS = (1, 16, 256, 4096)
_BETA = 0.25


def _vq_block_kernel(x_ref, w_ref, wn2_ref, wsq_ref, zq_ref, ids_ref, loss_ref):
    x = x_ref[...]
    bt = x.shape[0]
    slane = jax.lax.broadcasted_iota(jnp.int32, (_K, bt), 0)
    packed_row = jnp.zeros((1, bt), dtype=jnp.int32)
    loss = jnp.float32(0.0)
    for i in range(_NS):
        xi = x[:, i * _SD:(i + 1) * _SD]
        wi = w_ref[i]
        dot2 = jax.lax.dot_general(
            xi, wn2_ref[i], (((1,), (1,)), ((), ())),
            preferred_element_type=jnp.float32)
        flat_sq = jnp.sum(xi * xi, axis=1, keepdims=True)
        score = (flat_sq + wsq_ref[i][None, :]) + dot2
        score_t = jnp.transpose(score)
        smin = jnp.min(score_t, axis=0, keepdims=True)
        ids_row = jnp.min(jnp.where(score_t == smin, slane, _K),
                          axis=0, keepdims=True)
        onehot_t = (slane == ids_row).astype(jnp.float32)
        zq = jax.lax.dot_general(
            onehot_t, wi, (((0,), (0,)), ((), ())),
            preferred_element_type=jnp.float32)
        zq_ref[:, i * _SD:(i + 1) * _SD] = zq
        loss = loss + jnp.sum(smin)
        packed_row = packed_row + ids_row * jnp.int32(_OFFSETS[i])
    ids_ref[0, :, :] = packed_row
    loss_ref[...] = loss.reshape(1, 1, 1)


def kernel(h, W):
    Bq, Nq, Dq = h.shape
    tokens = Bq * Nq
    bt = 4096
    grid = tokens // bt
    hf = h.reshape(tokens, Dq)
    W_sq = jnp.sum(W * W, axis=2)
    W_n2 = W * jnp.float32(-2.0)

    zq, ids3, loss = pl.pallas_call(
        _vq_block_kernel,
        grid=(grid,),
        in_specs=[
            pl.BlockSpec((bt, Dq), lambda i: (i, 0)),
            pl.BlockSpec((_NS, _K, _SD), lambda i: (0, 0, 0)),
            pl.BlockSpec((_NS, _K, _SD), lambda i: (0, 0, 0)),
            pl.BlockSpec((_NS, _K), lambda i: (0, 0)),
        ],
        out_specs=[
            pl.BlockSpec((bt, Dq), lambda i: (i, 0)),
            pl.BlockSpec((1, 1, bt), lambda i: (i, 0, 0)),
            pl.BlockSpec((1, 1, 1), lambda i: (i, 0, 0)),
        ],
        out_shape=[
            jax.ShapeDtypeStruct((tokens, Dq), jnp.float32),
            jax.ShapeDtypeStruct((grid, 1, bt), jnp.int32),
            jax.ShapeDtypeStruct((grid, 1, 1), jnp.float32),
        ],
        compiler_params=pltpu.CompilerParams(
            dimension_semantics=("arbitrary",)),
    )(hf, W, W_n2, W_sq)

    z_q = zq.reshape(Bq, Nq, Dq)
    ids_packed = ids3.reshape(Bq, Nq)
    n_elems = jnp.float32(tokens * _SD)
    vq_total = (jnp.sum(loss) * jnp.float32(1.0 + _BETA)) / n_elems
    return (z_q, ids_packed, vq_total)
